# baseline (speedup 1.0000x reference)
import functools

import jax
import jax.numpy as jnp
from jax import lax
from jax.experimental import pallas as pl
from jax.experimental.pallas import tpu as pltpu
from jax.experimental.pallas import tpu_sc as plsc

HIDDEN = 4096
SEQ = 2048
N_CENT = 1024

NW = 32
ROWS_PER_TILE = SEQ // NW
CHUNK = 8
N_CHUNKS = ROWS_PER_TILE // CHUNK
LANES = 16
COL_CHUNKS = HIDDEN // LANES

CENT_BLOCK = 64
N_CENT_BLOCKS = N_CENT // CENT_BLOCK


N_BUF = 3


def _gather_sum_body(ids_hbm, table_hbm, out_hbm, idx_v, rows_v, acc_v,
                     sem0, sem1, sem2):
    wid = lax.axis_index("s") * 2 + lax.axis_index("c")
    pltpu.sync_copy(ids_hbm.at[pl.ds(wid * N_CHUNKS, N_CHUNKS)], idx_v)
    sems = (sem0, sem1, sem2)

    def start(c):
        buf = c % N_BUF
        return pltpu.async_copy(
            table_hbm.at[idx_v.at[c]], rows_v.at[buf], sems[buf])

    cps = [start(0), start(1), None]
    for c in range(N_CHUNKS):
        buf = c % N_BUF
        if c + 2 < N_CHUNKS:
            cps[(c + 2) % N_BUF] = start(c + 2)
        cps[buf].wait()

        if c == 0:
            @functools.partial(plsc.parallel_loop, 0, COL_CHUNKS, unroll=4)
            def _(j):
                col = pl.ds(j * LANES, LANES)
                s = rows_v[0, 0, col]
                for r in range(1, CHUNK):
                    s = s + rows_v[0, r, col]
                acc_v[col] = s
        else:
            @functools.partial(plsc.parallel_loop, 0, COL_CHUNKS, unroll=4)
            def _(j, _buf=buf):
                col = pl.ds(j * LANES, LANES)
                s = acc_v[col]
                for r in range(CHUNK):
                    s = s + rows_v[_buf, r, col]
                acc_v[col] = s

    pltpu.sync_copy(acc_v, out_hbm.at[wid])


@functools.cache
def _sc_gather_sum():
    return pl.kernel(
        _gather_sum_body,
        mesh=plsc.VectorSubcoreMesh(core_axis_name="c", subcore_axis_name="s"),
        out_type=jax.ShapeDtypeStruct((NW, HIDDEN), jnp.float32),
        scratch_types=[
            pltpu.VMEM((N_CHUNKS, CHUNK), jnp.int32),
            pltpu.VMEM((N_BUF, CHUNK, HIDDEN), jnp.float32),
            pltpu.VMEM((HIDDEN,), jnp.float32),
            pltpu.SemaphoreType.DMA,
            pltpu.SemaphoreType.DMA,
            pltpu.SemaphoreType.DMA,
        ],
    )


def _score_body(cid_ref, partials, cand, cent, vsurf, out, x_scr, min_scr):
    step = pl.program_id(0)

    @pl.when(step == 0)
    def _():
        sub = cid_ref[0] % 8
        mask = lax.broadcasted_iota(jnp.int32, (8, 1), 0) == sub
        cand_row = jnp.sum(jnp.where(mask, cand[...], 0.0), axis=0,
                           keepdims=True)
        xsum = jnp.sum(partials[...], axis=0, keepdims=True) + cand_row
        x_scr[...] = xsum * (1.0 / (SEQ + 1))
        min_scr[0] = jnp.float32(jnp.inf)

    x = x_scr[...]
    c = cent[...]
    ones = jnp.ones((1, HIDDEN), jnp.float32)
    nrm = lax.dot_general(c * c, ones, (((1,), (1,)), ((), ())),
                          precision=lax.Precision.HIGHEST,
                          preferred_element_type=jnp.float32)
    cx = lax.dot_general(c, x, (((1,), (1,)), ((), ())),
                         precision=lax.Precision.HIGHEST,
                         preferred_element_type=jnp.float32)
    min_scr[0] = jnp.minimum(min_scr[0], jnp.min(nrm - 2.0 * cx))

    @pl.when(step == N_CENT_BLOCKS - 1)
    def _():
        d2 = min_scr[0] + jnp.sum(x * x)
        dist = jnp.sqrt(d2)
        prox = 1.0 / (1.0 + dist)
        value = jnp.sum(vsurf[...] * x)
        vscore = 1.0 / (1.0 + jnp.exp(-value))
        score = jnp.exp(0.4 * jnp.log(prox) + 0.1 * jnp.log(vscore))
        out[...] = jnp.reshape(jnp.clip(score, 0.0, 1.0), (1, 1))


def kernel(context_ids, candidate_id, model_embeddings, reference_frame,
           attractor_centroids, value_surface):
    del reference_frame
    ids = context_ids.astype(jnp.int32).reshape(NW * N_CHUNKS, CHUNK)
    partials = _sc_gather_sum()(ids, model_embeddings)

    cid = jnp.asarray(candidate_id, jnp.int32).reshape(1)
    vsurf = value_surface.reshape(1, HIDDEN)

    score = pl.pallas_call(
        _score_body,
        grid_spec=pltpu.PrefetchScalarGridSpec(
            num_scalar_prefetch=1,
            grid=(N_CENT_BLOCKS,),
            in_specs=[
                pl.BlockSpec((NW, HIDDEN), lambda i, cid: (0, 0)),
                pl.BlockSpec((8, HIDDEN), lambda i, cid: (cid[0] // 8, 0)),
                pl.BlockSpec((CENT_BLOCK, HIDDEN), lambda i, cid: (i, 0)),
                pl.BlockSpec((1, HIDDEN), lambda i, cid: (0, 0)),
            ],
            out_specs=pl.BlockSpec((1, 1), lambda i, cid: (0, 0)),
            scratch_shapes=[
                pltpu.VMEM((1, HIDDEN), jnp.float32),
                pltpu.SMEM((1,), jnp.float32),
            ],
        ),
        out_shape=jax.ShapeDtypeStruct((1, 1), jnp.float32),
    )(cid, partials, model_embeddings, attractor_centroids, vsurf)
    return score[0, 0]


# speedup vs baseline: 1.1395x; 1.1395x over previous
import functools

import jax
import jax.numpy as jnp
from jax import lax
from jax.experimental import pallas as pl
from jax.experimental.pallas import tpu as pltpu
from jax.experimental.pallas import tpu_sc as plsc

HIDDEN = 4096
SEQ = 2048
N_CENT = 1024

NW = 32
ROWS_PER_TILE = SEQ // NW
CHUNK = 8
N_CHUNKS = ROWS_PER_TILE // CHUNK
LANES = 16
COL_CHUNKS = HIDDEN // LANES

CENT_BLOCK = 256
N_CENT_BLOCKS = N_CENT // CENT_BLOCK


N_BUF = 3


def _gather_sum_body(ids_hbm, table_hbm, out_hbm, idx_v, rows_v, acc_v,
                     sem0, sem1, sem2):
    wid = lax.axis_index("s") * 2 + lax.axis_index("c")
    pltpu.sync_copy(ids_hbm.at[pl.ds(wid * N_CHUNKS, N_CHUNKS)], idx_v)
    sems = (sem0, sem1, sem2)

    def start(c):
        buf = c % N_BUF
        return pltpu.async_copy(
            table_hbm.at[idx_v.at[c]], rows_v.at[buf], sems[buf])

    cps = [start(0), start(1), None]
    for c in range(N_CHUNKS):
        buf = c % N_BUF
        if c + 2 < N_CHUNKS:
            cps[(c + 2) % N_BUF] = start(c + 2)
        cps[buf].wait()

        if c == 0:
            @functools.partial(plsc.parallel_loop, 0, COL_CHUNKS, unroll=4)
            def _(j):
                col = pl.ds(j * LANES, LANES)
                s = rows_v[0, 0, col]
                for r in range(1, CHUNK):
                    s = s + rows_v[0, r, col]
                acc_v[col] = s
        else:
            @functools.partial(plsc.parallel_loop, 0, COL_CHUNKS, unroll=4)
            def _(j, _buf=buf):
                col = pl.ds(j * LANES, LANES)
                s = acc_v[col]
                for r in range(CHUNK):
                    s = s + rows_v[_buf, r, col]
                acc_v[col] = s

    pltpu.sync_copy(acc_v, out_hbm.at[wid])


@functools.cache
def _sc_gather_sum():
    return pl.kernel(
        _gather_sum_body,
        mesh=plsc.VectorSubcoreMesh(core_axis_name="c", subcore_axis_name="s"),
        out_type=jax.ShapeDtypeStruct((NW, HIDDEN), jnp.float32),
        scratch_types=[
            pltpu.VMEM((N_CHUNKS, CHUNK), jnp.int32),
            pltpu.VMEM((N_BUF, CHUNK, HIDDEN), jnp.float32),
            pltpu.VMEM((HIDDEN,), jnp.float32),
            pltpu.SemaphoreType.DMA,
            pltpu.SemaphoreType.DMA,
            pltpu.SemaphoreType.DMA,
        ],
    )


def _score_body(cid_ref, partials, cand, cent, vsurf, out, x_scr, min_scr):
    step = pl.program_id(0)

    @pl.when(step == 0)
    def _():
        sub = cid_ref[0] % 8
        mask = lax.broadcasted_iota(jnp.int32, (8, 1), 0) == sub
        cand_row = jnp.sum(jnp.where(mask, cand[...], 0.0), axis=0,
                           keepdims=True)
        xsum = jnp.sum(partials[...], axis=0, keepdims=True) + cand_row
        x_scr[...] = xsum * (1.0 / (SEQ + 1))
        min_scr[0] = jnp.float32(jnp.inf)

    x = x_scr[...]
    c = cent[...]
    ones = jnp.ones((1, HIDDEN), jnp.float32)
    nrm = lax.dot_general(c * c, ones, (((1,), (1,)), ((), ())),
                          precision=lax.Precision.HIGHEST,
                          preferred_element_type=jnp.float32)
    cx = lax.dot_general(c, x, (((1,), (1,)), ((), ())),
                         precision=lax.Precision.HIGHEST,
                         preferred_element_type=jnp.float32)
    min_scr[0] = jnp.minimum(min_scr[0], jnp.min(nrm - 2.0 * cx))

    @pl.when(step == N_CENT_BLOCKS - 1)
    def _():
        d2 = min_scr[0] + jnp.sum(x * x)
        dist = jnp.sqrt(d2)
        prox = 1.0 / (1.0 + dist)
        value = jnp.sum(vsurf[...] * x)
        vscore = 1.0 / (1.0 + jnp.exp(-value))
        score = jnp.exp(0.4 * jnp.log(prox) + 0.1 * jnp.log(vscore))
        out[...] = jnp.reshape(jnp.clip(score, 0.0, 1.0), (1, 1))


def kernel(context_ids, candidate_id, model_embeddings, reference_frame,
           attractor_centroids, value_surface):
    del reference_frame
    ids = context_ids.astype(jnp.int32).reshape(NW * N_CHUNKS, CHUNK)
    partials = _sc_gather_sum()(ids, model_embeddings)

    cid = jnp.asarray(candidate_id, jnp.int32).reshape(1)
    vsurf = value_surface.reshape(1, HIDDEN)

    score = pl.pallas_call(
        _score_body,
        grid_spec=pltpu.PrefetchScalarGridSpec(
            num_scalar_prefetch=1,
            grid=(N_CENT_BLOCKS,),
            in_specs=[
                pl.BlockSpec((NW, HIDDEN), lambda i, cid: (0, 0)),
                pl.BlockSpec((8, HIDDEN), lambda i, cid: (cid[0] // 8, 0)),
                pl.BlockSpec((CENT_BLOCK, HIDDEN), lambda i, cid: (i, 0)),
                pl.BlockSpec((1, HIDDEN), lambda i, cid: (0, 0)),
            ],
            out_specs=pl.BlockSpec((1, 1), lambda i, cid: (0, 0)),
            scratch_shapes=[
                pltpu.VMEM((1, HIDDEN), jnp.float32),
                pltpu.SMEM((1,), jnp.float32),
            ],
        ),
        out_shape=jax.ShapeDtypeStruct((1, 1), jnp.float32),
    )(cid, partials, model_embeddings, attractor_centroids, vsurf)
    return score[0, 0]


# speedup vs baseline: 1.1576x; 1.0159x over previous
import functools

import jax
import jax.numpy as jnp
from jax import lax
from jax.experimental import pallas as pl
from jax.experimental.pallas import tpu as pltpu
from jax.experimental.pallas import tpu_sc as plsc

HIDDEN = 4096
SEQ = 2048
N_CENT = 1024

NW = 32
ROWS_PER_TILE = SEQ // NW
CHUNK = 8
N_CHUNKS = ROWS_PER_TILE // CHUNK
LANES = 16
COL_CHUNKS = HIDDEN // LANES

CENT_BLOCK = 512
N_CENT_BLOCKS = N_CENT // CENT_BLOCK


N_BUF = 3


def _gather_sum_body(ids_hbm, table_hbm, out_hbm, idx_v, rows_v, acc_v,
                     sem0, sem1, sem2):
    wid = lax.axis_index("s") * 2 + lax.axis_index("c")
    pltpu.sync_copy(ids_hbm.at[pl.ds(wid * N_CHUNKS, N_CHUNKS)], idx_v)
    sems = (sem0, sem1, sem2)

    def start(c):
        buf = c % N_BUF
        return pltpu.async_copy(
            table_hbm.at[idx_v.at[c]], rows_v.at[buf], sems[buf])

    cps = [start(0), start(1), None]
    for c in range(N_CHUNKS):
        buf = c % N_BUF
        if c + 2 < N_CHUNKS:
            cps[(c + 2) % N_BUF] = start(c + 2)
        cps[buf].wait()

        if c == 0:
            @functools.partial(plsc.parallel_loop, 0, COL_CHUNKS, unroll=4)
            def _(j):
                col = pl.ds(j * LANES, LANES)
                s = rows_v[0, 0, col]
                for r in range(1, CHUNK):
                    s = s + rows_v[0, r, col]
                acc_v[col] = s
        else:
            @functools.partial(plsc.parallel_loop, 0, COL_CHUNKS, unroll=4)
            def _(j, _buf=buf):
                col = pl.ds(j * LANES, LANES)
                s = acc_v[col]
                for r in range(CHUNK):
                    s = s + rows_v[_buf, r, col]
                acc_v[col] = s

    pltpu.sync_copy(acc_v, out_hbm.at[wid])


@functools.cache
def _sc_gather_sum():
    return pl.kernel(
        _gather_sum_body,
        mesh=plsc.VectorSubcoreMesh(core_axis_name="c", subcore_axis_name="s"),
        out_type=jax.ShapeDtypeStruct((NW, HIDDEN), jnp.float32),
        scratch_types=[
            pltpu.VMEM((N_CHUNKS, CHUNK), jnp.int32),
            pltpu.VMEM((N_BUF, CHUNK, HIDDEN), jnp.float32),
            pltpu.VMEM((HIDDEN,), jnp.float32),
            pltpu.SemaphoreType.DMA,
            pltpu.SemaphoreType.DMA,
            pltpu.SemaphoreType.DMA,
        ],
    )


def _score_body(cid_ref, partials, cand, cent, vsurf, out, x_scr, min_scr):
    step = pl.program_id(0)

    @pl.when(step == 0)
    def _():
        sub = cid_ref[0] % 8
        mask = lax.broadcasted_iota(jnp.int32, (8, 1), 0) == sub
        cand_row = jnp.sum(jnp.where(mask, cand[...], 0.0), axis=0,
                           keepdims=True)
        xsum = jnp.sum(partials[...], axis=0, keepdims=True) + cand_row
        x_scr[...] = xsum * (1.0 / (SEQ + 1))
        min_scr[0] = jnp.float32(jnp.inf)

    x = x_scr[...]
    c = cent[...]
    ones = jnp.ones((1, HIDDEN), jnp.float32)
    nrm = lax.dot_general(c * c, ones, (((1,), (1,)), ((), ())),
                          precision=lax.Precision.HIGHEST,
                          preferred_element_type=jnp.float32)
    cx = lax.dot_general(c, x, (((1,), (1,)), ((), ())),
                         precision=lax.Precision.HIGHEST,
                         preferred_element_type=jnp.float32)
    min_scr[0] = jnp.minimum(min_scr[0], jnp.min(nrm - 2.0 * cx))

    @pl.when(step == N_CENT_BLOCKS - 1)
    def _():
        d2 = min_scr[0] + jnp.sum(x * x)
        dist = jnp.sqrt(d2)
        prox = 1.0 / (1.0 + dist)
        value = jnp.sum(vsurf[...] * x)
        vscore = 1.0 / (1.0 + jnp.exp(-value))
        score = jnp.exp(0.4 * jnp.log(prox) + 0.1 * jnp.log(vscore))
        out[...] = jnp.reshape(jnp.clip(score, 0.0, 1.0), (1, 1))


def kernel(context_ids, candidate_id, model_embeddings, reference_frame,
           attractor_centroids, value_surface):
    del reference_frame
    ids = context_ids.astype(jnp.int32).reshape(NW * N_CHUNKS, CHUNK)
    partials = _sc_gather_sum()(ids, model_embeddings)

    cid = jnp.asarray(candidate_id, jnp.int32).reshape(1)
    vsurf = value_surface.reshape(1, HIDDEN)

    score = pl.pallas_call(
        _score_body,
        grid_spec=pltpu.PrefetchScalarGridSpec(
            num_scalar_prefetch=1,
            grid=(N_CENT_BLOCKS,),
            in_specs=[
                pl.BlockSpec((NW, HIDDEN), lambda i, cid: (0, 0)),
                pl.BlockSpec((8, HIDDEN), lambda i, cid: (cid[0] // 8, 0)),
                pl.BlockSpec((CENT_BLOCK, HIDDEN), lambda i, cid: (i, 0)),
                pl.BlockSpec((1, HIDDEN), lambda i, cid: (0, 0)),
            ],
            out_specs=pl.BlockSpec((1, 1), lambda i, cid: (0, 0)),
            scratch_shapes=[
                pltpu.VMEM((1, HIDDEN), jnp.float32),
                pltpu.SMEM((1,), jnp.float32),
            ],
        ),
        out_shape=jax.ShapeDtypeStruct((1, 1), jnp.float32),
    )(cid, partials, model_embeddings, attractor_centroids, vsurf)
    return score[0, 0]
